# Initial kernel scaffold; baseline (speedup 1.0000x reference)
import jax
import jax.numpy as jnp
from jax.experimental import pallas as pl


def kernel(features, lyric, singer, genre, mid, W_feat, b_feat, E_sing, E_gen, E_mus, W_out, b_out):
    raise NotImplementedError("write your pallas kernel here")


# speedup vs baseline: 1.6884x; 1.6884x over previous
import functools

import jax
import jax.numpy as jnp
from jax import lax
from jax.experimental import pallas as pl
from jax.experimental.pallas import tpu as pltpu
from jax.experimental.pallas import tpu_sc as plsc

B = 16384
HID = 128
NC = 2
NS = 16
NW = NC * NS
BPW = B // NW
CH = 128
NCH = BPW // CH

_sc_mesh = plsc.VectorSubcoreMesh(core_axis_name="c", subcore_axis_name="s")


def _sc_gather_body(mid_h, sing_h, gen_h, emus_h, esing_h, egen_h,
                    out_m, out_s, out_g, idx_v, rows_v, sem):
    wid = lax.axis_index("s") * NC + lax.axis_index("c")
    for idx_h, tab_h, out_h in ((mid_h, emus_h, out_m),
                                (sing_h, esing_h, out_s),
                                (gen_h, egen_h, out_g)):
        pltpu.sync_copy(idx_h.at[pl.ds(wid * NCH, NCH)], idx_v)
        cps = [
            pltpu.async_copy(tab_h.at[idx_v.at[j]],
                             rows_v.at[pl.ds(j * CH, CH)], sem)
            for j in range(NCH)
        ]
        for cp in cps:
            cp.wait()
        pltpu.sync_copy(rows_v, out_h.at[pl.ds(wid * BPW, BPW)])


@functools.partial(
    pl.kernel,
    out_type=[jax.ShapeDtypeStruct((B, HID), jnp.float32)] * 3,
    mesh=_sc_mesh,
    scratch_types=[
        pltpu.VMEM((NCH, CH), jnp.int32),
        pltpu.VMEM((BPW, HID), jnp.float32),
        pltpu.SemaphoreType.DMA,
    ],
)
def _sc_gather(*args):
    _sc_gather_body(*args)


def _tc_body(feat_ref, memb_ref, sing_ref, gen_ref,
             wf_ref, bf_ref, wout_ref, bo_ref, out_ref):
    f = jnp.dot(feat_ref[:], wf_ref[:], preferred_element_type=jnp.float32)
    f = f + bf_ref[:]
    acc = jnp.dot(memb_ref[:], wout_ref[0:HID, :],
                  preferred_element_type=jnp.float32)
    acc = acc + jnp.dot(f, wout_ref[HID:2 * HID, :],
                        preferred_element_type=jnp.float32)
    acc = acc + jnp.dot(sing_ref[:], wout_ref[2 * HID:3 * HID, :],
                        preferred_element_type=jnp.float32)
    acc = acc + jnp.dot(gen_ref[:], wout_ref[3 * HID:4 * HID, :],
                        preferred_element_type=jnp.float32)
    out_ref[:] = acc + bo_ref[:]


def kernel(features, lyric, singer, genre, mid,
           W_feat, b_feat, E_sing, E_gen, E_mus, W_out, b_out):
    del lyric
    mid_i = mid.astype(jnp.int32).reshape(B // CH, CH)
    sing_i = singer.astype(jnp.int32).reshape(B // CH, CH)
    gen_i = genre.astype(jnp.int32).reshape(B // CH, CH)

    memb, sing, gen = _sc_gather(mid_i, sing_i, gen_i, E_mus, E_sing, E_gen)

    BLK = 1024
    grid = (B // BLK,)
    row_spec = pl.BlockSpec((BLK, HID), lambda i: (i, 0))
    out = pl.pallas_call(
        _tc_body,
        grid=grid,
        in_specs=[
            row_spec,
            row_spec,
            row_spec,
            row_spec,
            pl.BlockSpec((HID, HID), lambda i: (0, 0)),
            pl.BlockSpec((1, HID), lambda i: (0, 0)),
            pl.BlockSpec((4 * HID, 2 * HID), lambda i: (0, 0)),
            pl.BlockSpec((1, 2 * HID), lambda i: (0, 0)),
        ],
        out_specs=pl.BlockSpec((BLK, 2 * HID), lambda i: (i, 0)),
        out_shape=jax.ShapeDtypeStruct((B, 2 * HID), jnp.float32),
    )(features, memb, sing, gen,
      W_feat, b_feat.reshape(1, HID), W_out, b_out.reshape(1, 2 * HID))
    return out
